# baseline (speedup 1.0000x reference)
import functools

import jax
import jax.numpy as jnp
from jax import lax
from jax.experimental import pallas as pl
from jax.experimental.pallas import tpu as pltpu
from jax.experimental.pallas import tpu_sc as plsc

_L = 16
_NC = 2
_B = 16
_S = 2048
_D = 1024


def _sc_body(seq_hbm, mask_hbm, out_hbm, mask_v, row_v):
    b = lax.axis_index("s")
    pltpu.sync_copy(mask_hbm.at[b], mask_v.at[pl.ds(0, _S)])

    def _sum_step(i, acc):
        return acc + mask_v[pl.ds(i * _L, _L)]

    acc = lax.fori_loop(0, _S // _L, _sum_step,
                        jnp.zeros((_L,), jnp.int32), unroll=16)
    lanes = lax.iota(jnp.int32, _L)
    _dnums = lax.GatherDimensionNumbers(
        offset_dims=(), collapsed_slice_dims=(0,), start_index_map=(0,))
    for shift in (8, 4, 2, 1):
        rot = lax.gather(
            acc, ((lanes + shift) & (_L - 1))[:, None], _dnums, (1,),
            mode=lax.GatherScatterMode.PROMISE_IN_BOUNDS)
        acc = acc + rot
    total = acc[0]
    idx = jnp.maximum(total - 1, 0)

    mv = mask_v[pl.ds(idx, _L)][0]

    @pl.when(mv != 0)
    def _copy_row():
        pltpu.sync_copy(seq_hbm.at[b, idx], row_v)

    @pl.when(mv == 0)
    def _zero_row():
        z = jnp.zeros((_L,), jnp.float32)

        def _z_step(i, c):
            row_v[pl.ds(i * _L, _L)] = z
            return c

        lax.fori_loop(0, _D // _L, _z_step, 0, unroll=4)

    pltpu.sync_copy(row_v, out_hbm.at[b])


@jax.jit
def kernel(sequence, mask):
    mesh = plsc.VectorSubcoreMesh(core_axis_name="c", subcore_axis_name="s",
                                  num_cores=1)
    fn = pl.kernel(
        _sc_body,
        mesh=mesh,
        out_type=jax.ShapeDtypeStruct((_B, _D), jnp.float32),
        scratch_types=[
            pltpu.VMEM((_S + _L,), jnp.int32),
            pltpu.VMEM((_D,), jnp.float32),
        ],
    )
    return fn(sequence, mask)


# speedup vs baseline: 1.0028x; 1.0028x over previous
import functools

import jax
import jax.numpy as jnp
from jax import lax
from jax.experimental import pallas as pl
from jax.experimental.pallas import tpu as pltpu
from jax.experimental.pallas import tpu_sc as plsc

_L = 16
_NC = 2
_B = 16
_S = 2048
_D = 1024


def _sc_body(seq_hbm, mask_hbm, out_hbm, mask_v, row_v):
    b = lax.axis_index("s")
    pltpu.sync_copy(mask_hbm.at[b], mask_v.at[pl.ds(0, _S)])

    def _sum_step(i, acc):
        return acc + mask_v[pl.ds(i * _L, _L)]

    acc = lax.fori_loop(0, _S // _L, _sum_step,
                        jnp.zeros((_L,), jnp.int32), unroll=8)
    total = acc[0]
    for lane in range(1, _L):
        total = total + acc[lane]
    idx = jnp.maximum(total - 1, 0)

    mv = mask_v[pl.ds(idx, _L)][0]

    @pl.when(mv != 0)
    def _copy_row():
        pltpu.sync_copy(seq_hbm.at[b, idx], row_v)

    @pl.when(mv == 0)
    def _zero_row():
        z = jnp.zeros((_L,), jnp.float32)

        def _z_step(i, c):
            row_v[pl.ds(i * _L, _L)] = z
            return c

        lax.fori_loop(0, _D // _L, _z_step, 0, unroll=4)

    pltpu.sync_copy(row_v, out_hbm.at[b])


@jax.jit
def kernel(sequence, mask):
    mesh = plsc.VectorSubcoreMesh(core_axis_name="c", subcore_axis_name="s",
                                  num_cores=1)
    fn = pl.kernel(
        _sc_body,
        mesh=mesh,
        out_type=jax.ShapeDtypeStruct((_B, _D), jnp.float32),
        scratch_types=[
            pltpu.VMEM((_S + _L,), jnp.int32),
            pltpu.VMEM((_D,), jnp.float32),
        ],
    )
    return fn(sequence, mask)


# speedup vs baseline: 1.0272x; 1.0244x over previous
import jax
import jax.numpy as jnp
from jax import lax
from jax.experimental import pallas as pl
from jax.experimental.pallas import tpu as pltpu
from jax.experimental.pallas import tpu_sc as plsc
from jax._src.pallas import mpmd
from jax._src.pallas import core as pallas_core
from jax._src.pallas.mosaic import core as tpu_core

_L = 16
_B = 16
_S = 2048
_D = 1024

_vec_mesh = plsc.VectorSubcoreMesh(core_axis_name="c", subcore_axis_name="s",
                                   num_cores=1)
_scs_mesh = plsc.ScalarSubcoreMesh(axis_name="c", num_cores=1)

_TEC_VMEM = pallas_core.CoreMemorySpace(tpu_core.MemorySpace.VMEM, _vec_mesh)


def _scs_fn(seq_hbm, mask_hbm, out_hbm, mask_sh, scs_sem, done, mask_v,
            row_v):
    pltpu.async_copy(mask_hbm, mask_sh, scs_sem).wait()
    for t in range(_B):
        pl.semaphore_signal(done, 1, device_id={"s": t})


def _tec_fn(seq_hbm, mask_hbm, out_hbm, mask_sh, scs_sem, done, mask_v,
            row_v):
    b = lax.axis_index("s")
    pl.semaphore_wait(done, 1)
    pltpu.sync_copy(mask_sh.at[b], mask_v.at[pl.ds(0, _S)])

    def _sum_step(i, acc):
        return acc + mask_v[pl.ds(i * _L, _L)]

    acc = lax.fori_loop(0, _S // _L, _sum_step,
                        jnp.zeros((_L,), jnp.int32), unroll=8)
    total = acc[0]
    for lane in range(1, _L):
        total = total + acc[lane]
    idx = jnp.maximum(total - 1, 0)
    mv = mask_v[pl.ds(idx, _L)][0]

    @pl.when(mv != 0)
    def _copy_row():
        pltpu.sync_copy(seq_hbm.at[b, idx], row_v)

    @pl.when(mv == 0)
    def _zero_row():
        z = jnp.zeros((_L,), jnp.float32)

        def _z_step(i, c):
            row_v[pl.ds(i * _L, _L)] = z
            return c

        lax.fori_loop(0, _D // _L, _z_step, 0, unroll=4)

    pltpu.sync_copy(row_v, out_hbm.at[b])


@jax.jit
def kernel(sequence, mask):
    fn = mpmd.mpmd_map(
        [(_scs_mesh, _scs_fn), (_vec_mesh, _tec_fn)],
        out_types=jax.ShapeDtypeStruct((_B, _D), jnp.float32),
        scratch_types=[
            pltpu.MemorySpace.VMEM_SHARED((_B, _S), jnp.int32),
            pltpu.SemaphoreType.DMA @ _scs_mesh,
            pltpu.SemaphoreType.REGULAR @ _vec_mesh,
            _TEC_VMEM((_S + _L,), jnp.int32),
            _TEC_VMEM((_D,), jnp.float32),
        ],
    )
    return fn(sequence, mask)
